# Initial kernel scaffold; baseline (speedup 1.0000x reference)
import jax
import jax.numpy as jnp
from jax.experimental import pallas as pl


def kernel(input_ids, emb_table, ln0_weight, ln0_bias):
    raise NotImplementedError("write your pallas kernel here")


# speedup vs baseline: 1.0193x; 1.0193x over previous
import functools

import jax
import jax.numpy as jnp
from jax import lax
from jax.experimental import pallas as pl
from jax.experimental.pallas import tpu as pltpu
from jax.experimental.pallas import tpu_sc as plsc

VOCAB = 100000
D = 128
B = 4096
EPS = 1e-05

NC = 2
NS = 16
L = 16
NW = NC * NS
BPW = B // NW
NVR = D // L

_mesh = plsc.VectorSubcoreMesh(
    core_axis_name="c", subcore_axis_name="s", num_cores=NC, num_subcores=NS
)


@functools.partial(
    pl.kernel,
    out_type=jax.ShapeDtypeStruct((B, D), jnp.float32),
    mesh=_mesh,
    scratch_types=[
        pltpu.VMEM((BPW,), jnp.int32),
        pltpu.VMEM((BPW, D), jnp.float32),
        pltpu.VMEM((D,), jnp.float32),
        pltpu.VMEM((D,), jnp.float32),
        pltpu.SemaphoreType.DMA,
    ],
)
def _emb_ln_sc(idx_hbm, table_hbm, w_hbm, b_hbm, out_hbm,
               idx_v, rows_v, w_v, b_v, sem):
    wid = lax.axis_index("s") * NC + lax.axis_index("c")
    base = wid * BPW

    pltpu.sync_copy(idx_hbm.at[pl.ds(base, BPW)], idx_v)
    pltpu.sync_copy(w_hbm, w_v)
    pltpu.sync_copy(b_hbm, b_v)
    pltpu.async_copy(table_hbm.at[idx_v], rows_v, sem).wait()

    wv = [w_v[pl.ds(j * L, L)] for j in range(NVR)]
    bv = [b_v[pl.ds(j * L, L)] for j in range(NVR)]

    inv_d = jnp.float32(1.0 / D)
    perms = [(jnp.arange(L, dtype=jnp.int32) ^ k)[:, None] for k in (1, 2, 4, 8)]
    _dnums = lax.GatherDimensionNumbers(
        offset_dims=(), collapsed_slice_dims=(0,), start_index_map=(0,))

    def xlane_sum(t):
        for p in perms:
            t = t + lax.gather(
                t, p, _dnums, slice_sizes=(1,),
                mode=lax.GatherScatterMode.PROMISE_IN_BOUNDS)
        return t

    def row_body(r, carry):
        x = [rows_v[r, pl.ds(j * L, L)] for j in range(NVR)]
        s = x[0]
        q = x[0] * x[0]
        for j in range(1, NVR):
            s = s + x[j]
            q = q + x[j] * x[j]
        mean = xlane_sum(s) * inv_d
        var = xlane_sum(q) * inv_d - mean * mean
        v = var + jnp.float32(EPS)
        iv = lax.bitcast_convert_type(v, jnp.int32)
        y = lax.bitcast_convert_type(
            jnp.int32(0x5F3759DF) - lax.shift_right_arithmetic(iv, 1),
            jnp.float32)
        half_v = jnp.float32(0.5) * v
        for _ in range(3):
            y = y * (jnp.float32(1.5) - half_v * y * y)
        for j in range(NVR):
            rows_v[r, pl.ds(j * L, L)] = (x[j] - mean) * y * wv[j] + bv[j]
        return carry

    lax.fori_loop(0, BPW, row_body, jnp.int32(0))

    pltpu.sync_copy(rows_v, out_hbm.at[pl.ds(base, BPW)])


def kernel(input_ids, emb_table, ln0_weight, ln0_bias):
    idx = input_ids.reshape(B).astype(jnp.int32)
    return _emb_ln_sc(idx, emb_table, ln0_weight, ln0_bias)


# speedup vs baseline: 1.1251x; 1.1038x over previous
import functools

import jax
import jax.numpy as jnp
from jax import lax
from jax.experimental import pallas as pl
from jax.experimental.pallas import tpu as pltpu
from jax.experimental.pallas import tpu_sc as plsc

VOCAB = 100000
D = 128
B = 4096
EPS = 1e-05

NC = 2
NS = 16
L = 16
NW = NC * NS
BPW = B // NW
NVR = D // L

_mesh = plsc.VectorSubcoreMesh(
    core_axis_name="c", subcore_axis_name="s", num_cores=NC, num_subcores=NS
)


@functools.partial(
    pl.kernel,
    out_type=jax.ShapeDtypeStruct((B, D), jnp.float32),
    mesh=_mesh,
    scratch_types=[
        pltpu.VMEM((BPW,), jnp.int32),
        pltpu.VMEM((BPW, D), jnp.float32),
        pltpu.VMEM((D,), jnp.float32),
        pltpu.VMEM((D,), jnp.float32),
        pltpu.SemaphoreType.DMA,
    ],
)
def _emb_ln_sc(idx_hbm, table_hbm, w_hbm, b_hbm, out_hbm,
               idx_v, rows_v, w_v, b_v, sem):
    wid = lax.axis_index("s") * NC + lax.axis_index("c")
    base = wid * BPW

    pltpu.sync_copy(idx_hbm.at[pl.ds(base, BPW)], idx_v)
    pltpu.sync_copy(w_hbm, w_v)
    pltpu.sync_copy(b_hbm, b_v)
    pltpu.async_copy(table_hbm.at[idx_v], rows_v, sem).wait()

    wv = [w_v[pl.ds(j * L, L)] for j in range(NVR)]
    bv = [b_v[pl.ds(j * L, L)] for j in range(NVR)]

    inv_d = jnp.float32(1.0 / D)
    perms = [(jnp.arange(L, dtype=jnp.int32) ^ k)[:, None] for k in (1, 2, 4, 8)]
    _dnums = lax.GatherDimensionNumbers(
        offset_dims=(), collapsed_slice_dims=(0,), start_index_map=(0,))

    def xlane_sum(t):
        for p in perms:
            t = t + lax.gather(
                t, p, _dnums, slice_sizes=(1,),
                mode=lax.GatherScatterMode.PROMISE_IN_BOUNDS)
        return t

    def one_row(r):
        x = [rows_v[r, pl.ds(j * L, L)] for j in range(NVR)]
        s = x[0]
        q = x[0] * x[0]
        for j in range(1, NVR):
            s = s + x[j]
            q = q + x[j] * x[j]
        mean = xlane_sum(s) * inv_d
        var = xlane_sum(q) * inv_d - mean * mean
        v = var + jnp.float32(EPS)
        iv = lax.bitcast_convert_type(v, jnp.int32)
        y = lax.bitcast_convert_type(
            jnp.int32(0x5F3759DF) - lax.shift_right_arithmetic(iv, 1),
            jnp.float32)
        half_v = jnp.float32(0.5) * v
        for _ in range(2):
            y = y * (jnp.float32(1.5) - half_v * y * y)
        for j in range(NVR):
            rows_v[r, pl.ds(j * L, L)] = (x[j] - mean) * y * wv[j] + bv[j]

    ROWS_PER_IT = 4

    def row_body(i, carry):
        r0 = i * ROWS_PER_IT
        for k in range(ROWS_PER_IT):
            one_row(r0 + k)
        return carry

    lax.fori_loop(0, BPW // ROWS_PER_IT, row_body, jnp.int32(0))

    pltpu.sync_copy(rows_v, out_hbm.at[pl.ds(base, BPW)])


def kernel(input_ids, emb_table, ln0_weight, ln0_bias):
    idx = input_ids.reshape(B).astype(jnp.int32)
    return _emb_ln_sc(idx, emb_table, ln0_weight, ln0_bias)
